# Initial kernel scaffold; baseline (speedup 1.0000x reference)
import jax
import jax.numpy as jnp
from jax.experimental import pallas as pl


def kernel(x, edge_index, edge_label_index, Wl0, bl0, Wr0, gamma0, beta0, Wl1, bl1, Wr1, gamma1, beta1, Wl2, bl2, Wr2, gamma2, beta2):
    raise NotImplementedError("write your pallas kernel here")


# speedup vs baseline: 5.7540x; 5.7540x over previous
import dataclasses
import functools

import jax
import jax.numpy as jnp
from jax import lax
from jax.experimental import pallas as pl
from jax.experimental.pallas import tpu as pltpu
from jax.experimental.pallas import tpu_sc as plsc

NT = 32
TPS = 16
EC = 128
QC = 80


def _sc_mesh():
    return plsc.VectorSubcoreMesh(core_axis_name="c", subcore_axis_name="s")


def _make_sc_agg(N, D, E):
    nchunk = E // EC
    RC = 80
    nrc = N // RC

    @functools.partial(
        pl.kernel,
        out_type=jax.ShapeDtypeStruct((2, N, D), jnp.float32),
        mesh=_sc_mesh(),
        scratch_types=[
            pltpu.VMEM((EC,), jnp.int32),
            pltpu.VMEM((EC,), jnp.int32),
            pltpu.VMEM((EC, D), jnp.float32),
            pltpu.VMEM_SHARED((N, D), jnp.float32),
            pltpu.SemaphoreType.DMA,
        ])
    def sc_agg(h_hbm, src_hbm, dst_hbm, part_hbm, srcv, dstv, rows, accum,
               sem):
        c = lax.axis_index("c")
        s = lax.axis_index("s")
        wid = c * TPS + s

        @pl.loop(0, RC)
        def _(i):
            for j in range(0, D, 16):
                rows[i, pl.ds(j, 16)] = jnp.zeros((16,), jnp.float32)

        @pl.loop(s, nrc, step=TPS)
        def _(k):
            pltpu.sync_copy(rows.at[pl.ds(0, RC)], accum.at[pl.ds(k * RC, RC)])
        plsc.subcore_barrier()

        @pl.loop(wid, nchunk, step=NT)
        def _(j):
            base = j * EC
            pltpu.sync_copy(src_hbm.at[pl.ds(base, EC)], srcv)
            pltpu.sync_copy(dst_hbm.at[pl.ds(base, EC)], dstv)
            pltpu.async_copy(h_hbm.at[srcv], rows, sem).wait()
            pltpu.sync_copy(rows, accum.at[dstv], add=True)

        plsc.subcore_barrier()

        @pl.loop(s, nrc, step=TPS)
        def _(k):
            off = k * RC
            pltpu.sync_copy(accum.at[pl.ds(off, RC)],
                            part_hbm.at[c, pl.ds(off, RC)])

    return sc_agg


def _make_sc_cnt(N, E):
    nchunk = E // EC

    @functools.partial(
        pl.kernel,
        out_type=jax.ShapeDtypeStruct((NT * N,), jnp.float32),
        mesh=_sc_mesh(),
        scratch_types=[
            pltpu.VMEM((EC,), jnp.int32),
            pltpu.VMEM((N,), jnp.float32),
        ],
        compiler_params=dataclasses.replace(pltpu.CompilerParams(),
                                            needs_layout_passes=False))
    def sc_cnt(dst_hbm, cnt_hbm, dstv, hist):
        c = lax.axis_index("c")
        s = lax.axis_index("s")
        wid = c * TPS + s

        @pl.loop(0, N, step=16)
        def _(i):
            hist[pl.ds(i, 16)] = jnp.zeros((16,), jnp.float32)

        ones = jnp.ones((16,), jnp.float32)

        @pl.loop(wid, nchunk, step=NT)
        def _(j):
            pltpu.sync_copy(dst_hbm.at[pl.ds(j * EC, EC)], dstv)
            for g in range(EC // 16):
                plsc.addupdate_scatter(hist, [dstv[pl.ds(g * 16, 16)]], ones)

        pltpu.sync_copy(hist, cnt_hbm.at[pl.ds(wid * N, N)])

    return sc_cnt


def _make_sc_pred(N, D, Q):
    nchunk = Q // QC

    @functools.partial(
        pl.kernel,
        out_type=jax.ShapeDtypeStruct((Q, 16), jnp.float32),
        mesh=_sc_mesh(),
        scratch_types=[
            pltpu.VMEM((QC,), jnp.int32),
            pltpu.VMEM((QC,), jnp.int32),
            pltpu.VMEM((QC, D), jnp.float32),
            pltpu.VMEM((QC, D), jnp.float32),
            pltpu.VMEM((QC, 16), jnp.float32),
            pltpu.SemaphoreType.DMA,
        ])
    def sc_pred(h_hbm, qa_hbm, qb_hbm, pred_hbm, ia, ib, ra, rb, dots, sem):
        c = lax.axis_index("c")
        s = lax.axis_index("s")
        wid = c * TPS + s

        @pl.loop(wid, nchunk, step=NT)
        def _(j):
            base = j * QC
            pltpu.sync_copy(qa_hbm.at[pl.ds(base, QC)], ia)
            pltpu.sync_copy(qb_hbm.at[pl.ds(base, QC)], ib)
            pltpu.async_copy(h_hbm.at[ia], ra, sem).wait()
            pltpu.async_copy(h_hbm.at[ib], rb, sem).wait()

            @pl.loop(0, QC)
            def _(r):
                acc = ra[r, pl.ds(0, 16)] * rb[r, pl.ds(0, 16)]
                for k in range(1, D // 16):
                    acc = acc + ra[r, pl.ds(16 * k, 16)] * rb[r, pl.ds(16 * k, 16)]
                dots[r, :] = acc

            pltpu.sync_copy(dots, pred_hbm.at[pl.ds(base, QC)])

    return sc_pred


def _make_tc_rowsum(Q):
    def body(pp_ref, o_ref):
        o_ref[...] = jnp.sum(pp_ref[...], axis=1)

    return pl.pallas_call(body,
                          out_shape=jax.ShapeDtypeStruct((Q,), jnp.float32))


def _make_tc_layer(N, D, first, leaky):
    def body(h_ref, p_ref, ci_ref, wl_ref, bl_ref, wr_ref, g_ref, b_ref,
             o_ref, *inv_out):
        if first:
            cnt = jnp.sum(ci_ref[...], axis=1, keepdims=True)
            inv = 1.0 / jnp.maximum(cnt, 1.0)
            inv_out[0][...] = inv
        else:
            inv = ci_ref[...]
        agg = (p_ref[0] + p_ref[1]) * inv
        out = jnp.dot(agg, wl_ref[...], preferred_element_type=jnp.float32)
        out = out + jnp.dot(h_ref[...], wr_ref[...],
                            preferred_element_type=jnp.float32)
        out = out + bl_ref[...]
        nrm = jnp.sqrt(jnp.sum(out * out, axis=1, keepdims=True))
        out = out / jnp.maximum(nrm, 1e-12)
        m = jnp.mean(out, axis=0, keepdims=True)
        d = out - m
        v = jnp.mean(d * d, axis=0, keepdims=True)
        out = d * (g_ref[...] / jnp.sqrt(v + 1e-5)) + b_ref[...]
        if leaky:
            out = jnp.where(out > 0.0, out, 0.01 * out)
        o_ref[...] = out

    out_shape = [jax.ShapeDtypeStruct((N, D), jnp.float32)]
    if first:
        out_shape.append(jax.ShapeDtypeStruct((N, 1), jnp.float32))
    return pl.pallas_call(body, out_shape=out_shape)


def kernel(x, edge_index, edge_label_index, Wl0, bl0, Wr0, gamma0, beta0,
           Wl1, bl1, Wr1, gamma1, beta1, Wl2, bl2, Wr2, gamma2, beta2):
    N, D = x.shape
    E = edge_index.shape[1]
    Q = edge_label_index.shape[1]
    src, dst = edge_index[0], edge_index[1]
    qa, qb = edge_label_index[0], edge_label_index[1]

    sc_agg = _make_sc_agg(N, D, E)
    sc_cnt = _make_sc_cnt(N, E)
    sc_pred = _make_sc_pred(N, D, Q)

    params = [(Wl0, bl0, Wr0, gamma0, beta0), (Wl1, bl1, Wr1, gamma1, beta1),
              (Wl2, bl2, Wr2, gamma2, beta2)]

    cnt_t = sc_cnt(dst).reshape(NT, N).T
    h = x
    inv = None
    for i, (Wl, bl, Wr, g, b) in enumerate(params):
        parts = sc_agg(h, src, dst)
        ci = cnt_t if i == 0 else inv
        tc = _make_tc_layer(N, D, first=(i == 0), leaky=(i < 2))
        outs = tc(h, parts, ci, Wl, bl.reshape(1, D), Wr,
                  g.reshape(1, D), b.reshape(1, D))
        if i == 0:
            h, inv = outs
        else:
            h = outs[0]

    pp = sc_pred(h, qa, qb)
    pred = _make_tc_rowsum(Q)(pp)
    return (pred, h)
